# Initial kernel scaffold; baseline (speedup 1.0000x reference)
import jax
import jax.numpy as jnp
from jax.experimental import pallas as pl


def kernel(x, edge_weight, enc_W1, enc_b1, enc_W2, enc_b2, gcn_W, gcn_b, dec_W1, dec_b1, dec_W2, dec_b2, edge_index):
    raise NotImplementedError("write your pallas kernel here")


# speedup vs baseline: 9.8831x; 9.8831x over previous
import functools

import jax
import jax.numpy as jnp
from jax import lax
from jax.experimental import pallas as pl
from jax.experimental.pallas import tpu as pltpu
from jax.experimental.pallas import tpu_sc as plsc

N = 50000
E = 800000
HID = 64
L = 4

NSC = 2
NSUB = 16
HALF = 25088
NPAD = NSC * HALF
TRASH = 128
SROWS = HALF + TRASH
E_PAD = 802816
EPT = E_PAD // NSUB
BATCH = 1024
NBATCH = EPT // BATCH
CHUNK = 128
NCHUNK = BATCH // CHUNK

FH = 32
BM = 400
GRID = 125


def _vsplat(v, lane):
  idx = jnp.full((16, 1), lane, jnp.int32)
  return lax.gather(
      v, idx,
      lax.GatherDimensionNumbers(offset_dims=(), collapsed_slice_dims=(0,),
                                 start_index_map=(0,)),
      (1,), mode=lax.GatherScatterMode.PROMISE_IN_BOUNDS)


def _zero_vmem_2d(zb, nrows):
  def body(r, _):
    for f in range(FH // 16):
      zb[r, pl.ds(f * 16, 16)] = jnp.zeros((16,), jnp.float32)
    return 0
  lax.fori_loop(0, nrows, body, 0, unroll=False)


def _targets(colv, tcol, base):
  iota = lax.iota(jnp.int32, 16)
  for k in range(NCHUNK):
    for q in range(8):
      v = colv[pl.ds(k * CHUNK + q * 16, 16)]
      t = v - base
      m = (t >= 0) & (t < HALF)
      spread = HALF + ((iota + q * 16) & (TRASH - 1))
      tcol[k, pl.ds(q * 16, 16)] = jnp.where(m, t, spread)


def _deg_body(col_hbm, ew_hbm, out_hbm, colv, ewv, tcol, zb, dpart, sem):
  c = lax.axis_index("c")
  s = lax.axis_index("s")
  base = c * HALF

  per = SROWS // NSUB
  def zbody(r, _):
    zb[pl.ds(r * 16, 16)] = jnp.zeros((16,), jnp.float32)
    return 0
  lax.fori_loop(0, 99, zbody, 0)
  pltpu.sync_copy(zb.at[pl.ds(0, per)], dpart.at[pl.ds(s * per, per)])
  plsc.subcore_barrier()

  ebase = s * EPT
  def batch(b, _):
    eo = ebase + b * BATCH
    d1 = pltpu.async_copy(col_hbm.at[pl.ds(eo, BATCH)], colv, sem)
    d2 = pltpu.async_copy(ew_hbm.at[pl.ds(eo, BATCH)], ewv, sem)
    d1.wait()
    d2.wait()
    _targets(colv, tcol, base)
    for k in range(NCHUNK):
      pltpu.sync_copy(ewv.at[pl.ds(k * CHUNK, CHUNK)],
                      dpart.at[tcol.at[k]], add=True)
    return 0
  lax.fori_loop(0, NBATCH, batch, 0)
  plsc.subcore_barrier()

  outp = HALF // NSUB
  pltpu.sync_copy(dpart.at[pl.ds(s * outp, outp)], zb.at[pl.ds(0, outp)])
  pltpu.sync_copy(zb.at[pl.ds(0, outp)],
                  out_hbm.at[pl.ds(base + s * outp, outp)])


def _sc_deg(col_p, ew_p):
  mesh = plsc.VectorSubcoreMesh(core_axis_name="c", subcore_axis_name="s")
  kern = functools.partial(
      pl.kernel,
      mesh=mesh,
      compiler_params=pltpu.CompilerParams(use_tc_tiling_on_sc=False),
      out_type=jax.ShapeDtypeStruct((NPAD,), jnp.float32),
      scratch_types=[
          pltpu.VMEM((BATCH,), jnp.int32),
          pltpu.VMEM((BATCH,), jnp.float32),
          pltpu.VMEM((NCHUNK, CHUNK), jnp.int32),
          pltpu.VMEM((1584,), jnp.float32),
          pltpu.VMEM_SHARED((SROWS,), jnp.float32),
          pltpu.SemaphoreType.DMA,
      ],
  )(_deg_body)
  return kern(col_p, ew_p)


def _s_body(y_hbm, row_hbm, col_hbm, ew_hbm, out_hbm,
            rowv, colv, ewv, tcol, rows, zb, spart, sem, sem2):
  c = lax.axis_index("c")
  s = lax.axis_index("s")
  base = c * HALF

  _zero_vmem_2d(zb, 394)
  per = SROWS // NSUB
  for q in range(4):
    pltpu.sync_copy(zb, spart.at[pl.ds(s * per + q * 394, 394)])
  plsc.subcore_barrier()

  ebase = s * EPT
  def batch(b, _):
    eo = ebase + b * BATCH
    d1 = pltpu.async_copy(row_hbm.at[pl.ds(eo, BATCH)], rowv, sem)
    d2 = pltpu.async_copy(col_hbm.at[pl.ds(eo, BATCH)], colv, sem)
    d3 = pltpu.async_copy(ew_hbm.at[pl.ds(eo, BATCH)], ewv, sem)
    d1.wait()
    d2.wait()
    d3.wait()
    gs = [pltpu.async_copy(y_hbm.at[rowv.at[pl.ds(k * CHUNK, CHUNK)]],
                           rows.at[pl.ds(k * CHUNK, CHUNK)], sem2)
          for k in range(NCHUNK)]
    _targets(colv, tcol, base)
    for g in gs:
      g.wait()
    def scale(g, _):
      wv = ewv[pl.ds(g * 16, 16)]
      e0 = g * 16
      for ee in range(16):
        w = _vsplat(wv, ee)
        for f in range(FH // 16):
          rows[e0 + ee, pl.ds(f * 16, 16)] = (
              rows[e0 + ee, pl.ds(f * 16, 16)] * w)
      return 0
    lax.fori_loop(0, BATCH // 16, scale, 0)
    for k in range(NCHUNK):
      pltpu.sync_copy(rows.at[pl.ds(k * CHUNK, CHUNK)],
                      spart.at[tcol.at[k]], add=True)
    return 0
  lax.fori_loop(0, NBATCH, batch, 0)
  plsc.subcore_barrier()

  outp = HALF // NSUB
  for q in range(2):
    off = s * outp + q * 784
    pltpu.sync_copy(spart.at[pl.ds(off, 784)], rows.at[pl.ds(0, 784)])
    pltpu.sync_copy(rows.at[pl.ds(0, 784)],
                    out_hbm.at[pl.ds(base + off, 784)])


def _sc_scatter(y, row_p, col_p, ew_p):
  mesh = plsc.VectorSubcoreMesh(core_axis_name="c", subcore_axis_name="s")
  kern = functools.partial(
      pl.kernel,
      mesh=mesh,
      compiler_params=pltpu.CompilerParams(use_tc_tiling_on_sc=False),
      out_type=jax.ShapeDtypeStruct((NPAD, FH), jnp.float32),
      scratch_types=[
          pltpu.VMEM((BATCH,), jnp.int32),
          pltpu.VMEM((BATCH,), jnp.int32),
          pltpu.VMEM((BATCH,), jnp.float32),
          pltpu.VMEM((NCHUNK, CHUNK), jnp.int32),
          pltpu.VMEM((BATCH, FH), jnp.float32),
          pltpu.VMEM((394, FH), jnp.float32),
          pltpu.VMEM_SHARED((SROWS, FH), jnp.float32),
          pltpu.SemaphoreType.DMA,
          pltpu.SemaphoreType.DMA,
      ],
  )(_s_body)
  return kern(y, row_p, col_p, ew_p)


def _enc_body(x_ref, deg_ref, w1, b1, w2, b2, w0, y1_ref, y2_ref, dis_ref):
  xb = x_ref[...]
  h = jnp.maximum(jnp.dot(xb, w1[...], preferred_element_type=jnp.float32)
                  + b1[...], 0.0)
  h = jnp.dot(h, w2[...], preferred_element_type=jnp.float32) + b2[...]
  dis = lax.rsqrt(deg_ref[...] + 1.0)
  y = dis * jnp.dot(h, w0[...], preferred_element_type=jnp.float32)
  y1_ref[...] = y[:, :FH]
  y2_ref[...] = y[:, FH:]
  dis_ref[...] = dis


def _tc_enc(x, deg2d, w1, b1, w2, b2, w0):
  full = lambda shape: pl.BlockSpec(shape, lambda i: (0, 0))
  return pl.pallas_call(
      _enc_body,
      grid=(GRID,),
      in_specs=[
          pl.BlockSpec((BM, 2), lambda i: (i, 0)),
          pl.BlockSpec((BM, 1), lambda i: (i, 0)),
          full((2, HID)), full((1, HID)), full((HID, HID)), full((1, HID)),
          full((HID, HID)),
      ],
      out_specs=[
          pl.BlockSpec((BM, FH), lambda i: (i, 0)),
          pl.BlockSpec((BM, FH), lambda i: (i, 0)),
          pl.BlockSpec((BM, 1), lambda i: (i, 0)),
      ],
      out_shape=[
          jax.ShapeDtypeStruct((N, FH), jnp.float32),
          jax.ShapeDtypeStruct((N, FH), jnp.float32),
          jax.ShapeDtypeStruct((N, 1), jnp.float32),
      ],
  )(x, deg2d, w1, b1, w2, b2, w0)


def _layer_body(s1_ref, s2_ref, y1_ref, y2_ref, dis_ref, gb, w,
                o1_ref, o2_ref):
  dis = dis_ref[...]
  sy = jnp.concatenate([s1_ref[...] + y1_ref[...],
                        s2_ref[...] + y2_ref[...]], axis=1)
  h = dis * sy + gb[...]
  y = dis * jnp.dot(h, w[...], preferred_element_type=jnp.float32)
  o1_ref[...] = y[:, :FH]
  o2_ref[...] = y[:, FH:]


def _tc_layer(s1, s2, y1, y2, dis, gb, w):
  full = lambda shape: pl.BlockSpec(shape, lambda i: (0, 0))
  half = pl.BlockSpec((BM, FH), lambda i: (i, 0))
  return pl.pallas_call(
      _layer_body,
      grid=(GRID,),
      in_specs=[
          half, half, half, half,
          pl.BlockSpec((BM, 1), lambda i: (i, 0)),
          full((1, HID)), full((HID, HID)),
      ],
      out_specs=[half, half],
      out_shape=[
          jax.ShapeDtypeStruct((N, FH), jnp.float32),
          jax.ShapeDtypeStruct((N, FH), jnp.float32),
      ],
  )(s1, s2, y1, y2, dis, gb, w)


def _dec_body(s1_ref, s2_ref, y1_ref, y2_ref, dis_ref, gb, w1, b1, w2, b2,
              o_ref):
  dis = dis_ref[...]
  sy = jnp.concatenate([s1_ref[...] + y1_ref[...],
                        s2_ref[...] + y2_ref[...]], axis=1)
  h = dis * sy + gb[...]
  t = jnp.maximum(jnp.dot(h, w1[...], preferred_element_type=jnp.float32)
                  + b1[...], 0.0)
  o = jnp.dot(t, w2[...], preferred_element_type=jnp.float32) + b2[...]
  o_ref[...] = jax.nn.sigmoid(o)


def _tc_dec(s1, s2, y1, y2, dis, gb, w1, b1, w2, b2):
  full = lambda shape: pl.BlockSpec(shape, lambda i: (0, 0))
  half = pl.BlockSpec((BM, FH), lambda i: (i, 0))
  return pl.pallas_call(
      _dec_body,
      grid=(GRID,),
      in_specs=[
          half, half, half, half,
          pl.BlockSpec((BM, 1), lambda i: (i, 0)),
          full((1, HID)), full((HID, HID)), full((1, HID)),
          full((HID, 1)), full((1, 1)),
      ],
      out_specs=pl.BlockSpec((BM, 1), lambda i: (i, 0)),
      out_shape=jax.ShapeDtypeStruct((N, 1), jnp.float32),
  )(s1, s2, y1, y2, dis, gb, w1, b1, w2, b2)


def kernel(x, edge_weight, enc_W1, enc_b1, enc_W2, enc_b2, gcn_W, gcn_b,
           dec_W1, dec_b1, dec_W2, dec_b2, edge_index):
  row = edge_index[0].astype(jnp.int32)
  col = edge_index[1].astype(jnp.int32)
  pad = E_PAD - E
  row_p = jnp.concatenate([row, jnp.zeros((pad,), jnp.int32)])
  col_p = jnp.concatenate([col, jnp.zeros((pad,), jnp.int32)])
  ew_p = jnp.concatenate([edge_weight, jnp.zeros((pad,), jnp.float32)])

  deg = _sc_deg(col_p, ew_p)
  deg2d = deg[:N].reshape(N, 1)

  b1 = enc_b1.reshape(1, HID)
  b2 = enc_b2.reshape(1, HID)
  y1, y2, dis = _tc_enc(x, deg2d, enc_W1, b1, enc_W2, b2, gcn_W[0])

  for i in range(L):
    s1 = _sc_scatter(y1, row_p, col_p, ew_p)
    s2 = _sc_scatter(y2, row_p, col_p, ew_p)
    gb = gcn_b[i].reshape(1, HID)
    if i < L - 1:
      y1, y2 = _tc_layer(s1, s2, y1, y2, dis, gb, gcn_W[i + 1])
    else:
      out = _tc_dec(s1, s2, y1, y2, dis, gb, dec_W1, dec_b1.reshape(1, HID),
                    dec_W2, dec_b2.reshape(1, 1))
  return out
